# Initial kernel scaffold; baseline (speedup 1.0000x reference)
import jax
import jax.numpy as jnp
from jax.experimental import pallas as pl


def kernel(x, edge_index, W1, b1, W2, b2, M1, c1, M2, c2):
    raise NotImplementedError("write your pallas kernel here")


# speedup vs baseline: 6.4674x; 6.4674x over previous
import functools

import jax
import jax.numpy as jnp
from jax import lax
from jax.experimental import pallas as pl
from jax.experimental.pallas import tpu as pltpu
from jax.experimental.pallas import tpu_sc as plsc

N = 10000
E = 320000
DIN = 128
DH = 256
DOUT = 2
NC = 2
NS = 16
NW = NC * NS
HALF = DH // 2

EPW = E // NW
EPW_FC = EPW // 128
EPW_T = EPW - EPW_FC * 128
EPS = E // NS
EPS_FC = EPS // 128
EPS_T = EPS - EPS_FC * 128
RPW = 624
RT = N - NS * RPW

_F32 = jnp.float32


def _sc_mesh():
    return plsc.VectorSubcoreMesh(core_axis_name="c", subcore_axis_name="s")


_DCH = 2000


def _deg_call(dst_hbm):
    @functools.partial(
        pl.kernel,
        out_type=jax.ShapeDtypeStruct((NW * N,), _F32),
        mesh=_sc_mesh(),
        compiler_params=pltpu.CompilerParams(needs_layout_passes=False),
        scratch_types=[
            pltpu.VMEM((N,), _F32),
            pltpu.VMEM((_DCH,), jnp.int32),
        ],
    )
    def deg_k(dst_ref, out_ref, hist, dbuf):
        c = lax.axis_index("c")
        s = lax.axis_index("s")
        w = c * NS + s
        z = jnp.zeros((16,), _F32)
        one = jnp.ones((16,), _F32)

        def zbody(i, carry):
            hist[pl.ds(i * 16, 16)] = z
            return carry

        lax.fori_loop(0, N // 16, zbody, 0)
        base = w * EPW

        def chunk(ci, carry):
            off = pl.multiple_of(base + ci * _DCH, 16)
            pltpu.sync_copy(dst_ref.at[pl.ds(off, _DCH)], dbuf)

            def gbody(g, carry2):
                dv = dbuf[pl.ds(g * 16, 16)]
                plsc.addupdate_scatter(hist, [dv], one)
                return carry2

            lax.fori_loop(0, _DCH // 16, gbody, 0)
            return carry

        lax.fori_loop(0, EPW // _DCH, chunk, 0)
        pltpu.sync_copy(hist, out_ref.at[pl.ds(w * N, N)])

    return deg_k(dst_hbm)


def _tc_dinv(degp):
    def body(d_ref, o_ref):
        deg = jnp.sum(d_ref[...], axis=0, keepdims=True) + 1.0
        dinv = lax.rsqrt(deg)
        o_ref[...] = jnp.broadcast_to(jnp.transpose(dinv), (N, DH))

    return pl.pallas_call(
        body,
        out_shape=jax.ShapeDtypeStruct((N, DH), _F32),
    )(degp)


def _scatter_call(gflat, src2, dst):
    @functools.partial(
        pl.kernel,
        out_type=jax.ShapeDtypeStruct((NC * N, HALF), _F32),
        mesh=_sc_mesh(),
        compiler_params=pltpu.CompilerParams(needs_layout_passes=False),
        scratch_types=[
            pltpu.VMEM_SHARED((N, HALF), _F32),
            pltpu.VMEM((128,), jnp.int32),
            pltpu.VMEM((128,), jnp.int32),
            pltpu.VMEM((128, HALF), _F32),
            pltpu.VMEM((EPS_T,), jnp.int32),
            pltpu.VMEM((EPS_T,), jnp.int32),
            pltpu.VMEM((EPS_T, HALF), _F32),
        ],
    )
    def scat_k(g_ref, src_ref, dst_ref, out_ref,
               acc, idx_s, idx_d, rows, idx_st, idx_dt, rows_t):
        c = lax.axis_index("c")
        s = lax.axis_index("s")
        r0 = s * RPW
        pltpu.sync_copy(g_ref.at[pl.ds(c * N + r0, RPW)], acc.at[pl.ds(r0, RPW)])

        @pl.when(s == 0)
        def _():
            pltpu.sync_copy(g_ref.at[pl.ds(c * N + NS * RPW, RT)],
                            acc.at[pl.ds(NS * RPW, RT)])

        plsc.subcore_barrier()
        ebase = s * EPS

        def body(i, carry):
            off = pl.multiple_of(ebase + i * 128, 32)
            pltpu.sync_copy(src_ref.at[pl.ds(c * E + off, 128)], idx_s)
            pltpu.sync_copy(g_ref.at[idx_s], rows)
            pltpu.sync_copy(dst_ref.at[pl.ds(off, 128)], idx_d)
            pltpu.sync_copy(rows, acc.at[idx_d], add=True)
            return carry

        lax.fori_loop(0, EPS_FC, body, 0)
        off_t = ebase + EPS_FC * 128
        pltpu.sync_copy(src_ref.at[pl.ds(c * E + off_t, EPS_T)], idx_st)
        pltpu.sync_copy(g_ref.at[idx_st], rows_t)
        pltpu.sync_copy(dst_ref.at[pl.ds(off_t, EPS_T)], idx_dt)
        pltpu.sync_copy(rows_t, acc.at[idx_dt], add=True)
        plsc.subcore_barrier()
        pltpu.sync_copy(acc.at[pl.ds(r0, RPW)],
                        out_ref.at[pl.ds(c * N + r0, RPW)])

        @pl.when(s == 0)
        def _():
            pltpu.sync_copy(acc.at[pl.ds(NS * RPW, RT)],
                            out_ref.at[pl.ds(c * N + NS * RPW, RT)])

    return scat_k(gflat, src2, dst)


def _edge_call(p_hbm, q_hbm, src, dst, m2t, c2pad):
    @functools.partial(
        pl.kernel,
        out_type=jax.ShapeDtypeStruct((E * DOUT,), _F32),
        mesh=_sc_mesh(),
        compiler_params=pltpu.CompilerParams(needs_layout_passes=False),
        scratch_types=[
            pltpu.VMEM((128,), jnp.int32),
            pltpu.VMEM((128,), jnp.int32),
            pltpu.VMEM((128, DH), _F32),
            pltpu.VMEM((128, DH), _F32),
            pltpu.VMEM((128 * DOUT,), _F32),
            pltpu.VMEM((EPW_T,), jnp.int32),
            pltpu.VMEM((EPW_T,), jnp.int32),
            pltpu.VMEM((EPW_T, DH), _F32),
            pltpu.VMEM((EPW_T, DH), _F32),
            pltpu.VMEM((EPW_T * DOUT,), _F32),
            pltpu.VMEM((DOUT, DH), _F32),
            pltpu.VMEM((16,), _F32),
        ],
    )
    def edge_k(p_ref, q_ref, src_ref, dst_ref, m2_ref, c2_ref, out_ref,
               idx_s, idx_d, pb, qb, ob,
               idx_st, idx_dt, pbt, qbt, obt, m2v, c2v):
        c = lax.axis_index("c")
        s = lax.axis_index("s")
        w = c * NS + s
        pltpu.sync_copy(m2_ref, m2v)
        pltpu.sync_copy(c2_ref, c2v)
        m0 = [m2v[0, pl.ds(k * 16, 16)] for k in range(DH // 16)]
        m1 = [m2v[1, pl.ds(k * 16, 16)] for k in range(DH // 16)]
        c2vec = c2v[...]
        c2a = c2vec[0]
        c2b = c2vec[1]
        iot = lax.iota(jnp.int32, 16)
        base = w * EPW

        def do_chunk(off, n, isx, idx, pbuf, qbuf, obuf):
            pltpu.sync_copy(src_ref.at[pl.ds(off, n)], isx)
            pltpu.sync_copy(p_ref.at[isx], pbuf)
            pltpu.sync_copy(dst_ref.at[pl.ds(off, n)], idx)
            pltpu.sync_copy(q_ref.at[idx], qbuf)

            def ebody(e, carry):
                acc0 = jnp.zeros((16,), _F32)
                acc1 = jnp.zeros((16,), _F32)
                for k in range(DH // 16):
                    pv = pbuf[e, pl.ds(k * 16, 16)]
                    qv = qbuf[e, pl.ds(k * 16, 16)]
                    r = jnp.maximum(pv + qv, 0.0)
                    acc0 = acc0 + r * m0[k]
                    acc1 = acc1 + r * m1[k]
                s0 = jnp.sum(acc0) + c2a
                s1 = jnp.sum(acc1) + c2b
                vvals = jnp.where(iot == 0, s0, s1)
                vidx = e * DOUT + jnp.minimum(iot, 1)
                plsc.store_scatter(obuf, [vidx], vvals, mask=iot < DOUT)
                return carry

            lax.fori_loop(0, n, ebody, 0)
            pltpu.sync_copy(obuf, out_ref.at[pl.ds(off * DOUT, n * DOUT)])

        def body(i, carry):
            off = pl.multiple_of(base + i * 128, 16)
            do_chunk(off, 128, idx_s, idx_d, pb, qb, ob)
            return carry

        lax.fori_loop(0, EPW_FC, body, 0)
        do_chunk(base + EPW_FC * 128, EPW_T, idx_st, idx_dt, pbt, qbt, obt)

    return edge_k(p_hbm, q_hbm, src, dst, m2t, c2pad)


_RB = 2000
_GRID = N // _RB


def _tc_a(x, dinv_bc, w1):
    def body(x_ref, d_ref, w_ref, g_ref):
        h = jnp.dot(x_ref[...], w_ref[...], preferred_element_type=_F32)
        g = d_ref[...] * h
        g_ref[0] = g[:, :HALF]
        g_ref[1] = g[:, HALF:]

    return pl.pallas_call(
        body,
        grid=(_GRID,),
        in_specs=[
            pl.BlockSpec((_RB, DIN), lambda i: (i, 0)),
            pl.BlockSpec((_RB, DH), lambda i: (i, 0)),
            pl.BlockSpec((DIN, DH), lambda i: (0, 0)),
        ],
        out_specs=pl.BlockSpec((NC, _RB, HALF), lambda i: (0, i, 0)),
        out_shape=jax.ShapeDtypeStruct((NC, N, HALF), _F32),
    )(x, dinv_bc, w1)


def _tc_b(s1, dinv_bc, b1, w2):
    def body(s_ref, d_ref, b_ref, w_ref, g_ref):
        scat = jnp.concatenate([s_ref[0], s_ref[1]], axis=1)
        act = jnp.maximum(d_ref[...] * scat + b_ref[0, :], 0.0)
        h2 = jnp.dot(act, w_ref[...], preferred_element_type=_F32)
        g = d_ref[...] * h2
        g_ref[0] = g[:, :HALF]
        g_ref[1] = g[:, HALF:]

    return pl.pallas_call(
        body,
        grid=(_GRID,),
        in_specs=[
            pl.BlockSpec((NC, _RB, HALF), lambda i: (0, i, 0)),
            pl.BlockSpec((_RB, DH), lambda i: (i, 0)),
            pl.BlockSpec((1, DH), lambda i: (0, 0)),
            pl.BlockSpec((DH, DH), lambda i: (0, 0)),
        ],
        out_specs=pl.BlockSpec((NC, _RB, HALF), lambda i: (0, i, 0)),
        out_shape=jax.ShapeDtypeStruct((NC, N, HALF), _F32),
    )(s1, dinv_bc, b1, w2)


def _tc_c(s2, dinv_bc, b2, m1, c1):
    def body(s_ref, d_ref, b_ref, m_ref, c_ref, p_ref, q_ref):
        scat = jnp.concatenate([s_ref[0], s_ref[1]], axis=1)
        h = d_ref[...] * scat + b_ref[0, :]
        p_ref[...] = jnp.dot(h, m_ref[:DH, :],
                             preferred_element_type=_F32) + c_ref[0, :]
        q_ref[...] = jnp.dot(h, m_ref[DH:, :], preferred_element_type=_F32)

    return pl.pallas_call(
        body,
        grid=(_GRID,),
        in_specs=[
            pl.BlockSpec((NC, _RB, HALF), lambda i: (0, i, 0)),
            pl.BlockSpec((_RB, DH), lambda i: (i, 0)),
            pl.BlockSpec((1, DH), lambda i: (0, 0)),
            pl.BlockSpec((2 * DH, DH), lambda i: (0, 0)),
            pl.BlockSpec((1, DH), lambda i: (0, 0)),
        ],
        out_specs=[
            pl.BlockSpec((_RB, DH), lambda i: (i, 0)),
            pl.BlockSpec((_RB, DH), lambda i: (i, 0)),
        ],
        out_shape=[
            jax.ShapeDtypeStruct((N, DH), _F32),
            jax.ShapeDtypeStruct((N, DH), _F32),
        ],
    )(s2, dinv_bc, b2, m1, c1)


def kernel(x, edge_index, W1, b1, W2, b2, M1, c1, M2, c2):
    ei = edge_index.astype(jnp.int32)
    src = ei[0]
    dst = ei[1]
    src2 = jnp.concatenate([src, src + N])
    b1r = b1.reshape(1, DH)
    b2r = b2.reshape(1, DH)
    c1r = c1.reshape(1, DH)
    m2t = M2.T.astype(_F32)
    c2pad = jnp.concatenate([c2, jnp.zeros((14,), _F32)])

    degp = _deg_call(dst).reshape(NW, N)
    dinv_bc = _tc_dinv(degp)
    g1 = _tc_a(x, dinv_bc, W1)
    s1 = _scatter_call(g1.reshape(NC * N, HALF), src2, dst)
    g2 = _tc_b(s1.reshape(NC, N, HALF), dinv_bc, b1r, W2)
    s2 = _scatter_call(g2.reshape(NC * N, HALF), src2, dst)
    p, q = _tc_c(s2.reshape(NC, N, HALF), dinv_bc, b2r, M1, c1r)
    out = _edge_call(p, q, src, dst, m2t, c2pad)
    return out.reshape(E, DOUT)


# speedup vs baseline: 10.5469x; 1.6308x over previous
import functools

import jax
import jax.numpy as jnp
from jax import lax
from jax.experimental import pallas as pl
from jax.experimental.pallas import tpu as pltpu
from jax.experimental.pallas import tpu_sc as plsc

N = 10000
E = 320000
DIN = 128
DH = 256
DOUT = 2
NC = 2
NS = 16
NW = NC * NS
HALF = DH // 2

EPW = E // NW
ECH = 96
E_FC = 104
EPW_T = EPW - E_FC * ECH
EPS = E // NS
EPS_FC = EPS // 128
EPS_T = EPS - EPS_FC * 128
RPW = 624
RT = N - NS * RPW

_F32 = jnp.float32


def _sc_mesh():
    return plsc.VectorSubcoreMesh(core_axis_name="c", subcore_axis_name="s")


_DCH = 2000


def _deg_call(dst_hbm):
    @functools.partial(
        pl.kernel,
        out_type=jax.ShapeDtypeStruct((NW * N,), _F32),
        mesh=_sc_mesh(),
        compiler_params=pltpu.CompilerParams(needs_layout_passes=False),
        scratch_types=[
            pltpu.VMEM((N,), _F32),
            pltpu.VMEM((_DCH,), jnp.int32),
        ],
    )
    def deg_k(dst_ref, out_ref, hist, dbuf):
        c = lax.axis_index("c")
        s = lax.axis_index("s")
        w = c * NS + s
        z = jnp.zeros((16,), _F32)
        one = jnp.ones((16,), _F32)

        def zbody(i, carry):
            hist[pl.ds(i * 16, 16)] = z
            return carry

        lax.fori_loop(0, N // 16, zbody, 0)
        base = w * EPW

        def chunk(ci, carry):
            off = pl.multiple_of(base + ci * _DCH, 16)
            pltpu.sync_copy(dst_ref.at[pl.ds(off, _DCH)], dbuf)

            def gbody(g, carry2):
                dv = dbuf[pl.ds(g * 16, 16)]
                plsc.addupdate_scatter(hist, [dv], one)
                return carry2

            lax.fori_loop(0, _DCH // 16, gbody, 0)
            return carry

        lax.fori_loop(0, EPW // _DCH, chunk, 0)
        pltpu.sync_copy(hist, out_ref.at[pl.ds(w * N, N)])

    return deg_k(dst_hbm)


def _tc_dinv(degp):
    def body(d_ref, o_ref):
        deg = jnp.sum(d_ref[...], axis=0, keepdims=True) + 1.0
        dinv = lax.rsqrt(deg)
        o_ref[...] = jnp.broadcast_to(jnp.transpose(dinv), (N, DH))

    return pl.pallas_call(
        body,
        out_shape=jax.ShapeDtypeStruct((N, DH), _F32),
    )(degp)


def _scatter_call(gflat, src2, dst):
    @functools.partial(
        pl.kernel,
        out_type=jax.ShapeDtypeStruct((NC * N, HALF), _F32),
        mesh=_sc_mesh(),
        compiler_params=pltpu.CompilerParams(needs_layout_passes=False),
        scratch_types=[
            pltpu.VMEM_SHARED((N, HALF), _F32),
            pltpu.VMEM((2, 128), jnp.int32),
            pltpu.VMEM((2, 128), jnp.int32),
            pltpu.VMEM((2, 128), jnp.int32),
            pltpu.VMEM((2, 128), jnp.int32),
            pltpu.VMEM((128, HALF), _F32),
            pltpu.VMEM((128, HALF), _F32),
            pltpu.VMEM((EPS_T,), jnp.int32),
            pltpu.VMEM((EPS_T,), jnp.int32),
            pltpu.VMEM((EPS_T, HALF), _F32),
            pltpu.SemaphoreType.DMA,
            pltpu.SemaphoreType.DMA,
            pltpu.SemaphoreType.DMA,
            pltpu.SemaphoreType.DMA,
            pltpu.SemaphoreType.DMA,
            pltpu.SemaphoreType.DMA,
            pltpu.SemaphoreType.DMA,
        ],
    )
    def scat_k(g_ref, src_ref, dst_ref, out_ref,
               acc, idx0, idx1, idx2, idx3, rows0, rows1,
               idx_st, idx_dt, rows_t,
               semi0, semi1, semi2, semi3, semg, sems0, sems1):
        c = lax.axis_index("c")
        s = lax.axis_index("s")
        r0 = s * RPW
        idx_bufs = [idx0, idx1, idx2, idx3]
        semi = [semi0, semi1, semi2, semi3]
        rows = [rows0, rows1]
        sems = [sems0, sems1]
        ebase = s * EPS

        def issue_idx(i, slot):
            off = pl.multiple_of(ebase + i * 128, 32)
            pltpu.async_copy(src_ref.at[pl.ds(c * E + off, 128)],
                             idx_bufs[slot].at[0], semi[slot])
            pltpu.async_copy(dst_ref.at[pl.ds(off, 128)],
                             idx_bufs[slot].at[1], semi[slot])

        def drain_idx(slot):
            pltpu.make_async_copy(dst_ref.at[pl.ds(0, 128)],
                                  idx_bufs[slot].at[0], semi[slot]).wait()
            pltpu.make_async_copy(dst_ref.at[pl.ds(0, 128)],
                                  idx_bufs[slot].at[1], semi[slot]).wait()

        pltpu.sync_copy(g_ref.at[pl.ds(c * N + r0, RPW)], acc.at[pl.ds(r0, RPW)])

        @pl.when(s == 0)
        def _():
            pltpu.sync_copy(g_ref.at[pl.ds(c * N + NS * RPW, RT)],
                            acc.at[pl.ds(NS * RPW, RT)])

        plsc.subcore_barrier()
        issue_idx(0, 0)
        issue_idx(1, 1)

        def body(j, carry):
            for b in range(4):
                i = j * 4 + b

                @pl.when(i >= 2)
                def _():
                    pltpu.make_async_copy(g_ref.at[pl.ds(0, 128)],
                                          rows[b % 2], sems[b % 2]).wait()

                @pl.when(i + 2 < EPS_FC)
                def _():
                    issue_idx(i + 2, (b + 2) % 4)

                drain_idx(b)
                pltpu.async_copy(g_ref.at[idx_bufs[b].at[0]], rows[b % 2],
                                 semg).wait()
                pltpu.async_copy(rows[b % 2], acc.at[idx_bufs[b].at[1]],
                                 sems[b % 2], add=True)
            return carry

        lax.fori_loop(0, EPS_FC // 4, body, 0)
        pltpu.make_async_copy(g_ref.at[pl.ds(0, 128)], rows[0], sems[0]).wait()
        pltpu.make_async_copy(g_ref.at[pl.ds(0, 128)], rows[1], sems[1]).wait()
        off_t = ebase + EPS_FC * 128
        pltpu.sync_copy(src_ref.at[pl.ds(c * E + off_t, EPS_T)], idx_st)
        pltpu.sync_copy(g_ref.at[idx_st], rows_t)
        pltpu.sync_copy(dst_ref.at[pl.ds(off_t, EPS_T)], idx_dt)
        pltpu.sync_copy(rows_t, acc.at[idx_dt], add=True)
        plsc.subcore_barrier()
        pltpu.sync_copy(acc.at[pl.ds(r0, RPW)],
                        out_ref.at[pl.ds(c * N + r0, RPW)])

        @pl.when(s == 0)
        def _():
            pltpu.sync_copy(acc.at[pl.ds(NS * RPW, RT)],
                            out_ref.at[pl.ds(c * N + NS * RPW, RT)])

    return scat_k(gflat, src2, dst)


def _edge_call(p_hbm, q_hbm, src, dst, m2t, c2pad):
    @functools.partial(
        pl.kernel,
        out_type=jax.ShapeDtypeStruct((E * DOUT,), _F32),
        mesh=_sc_mesh(),
        compiler_params=pltpu.CompilerParams(needs_layout_passes=False),
        scratch_types=[
            pltpu.VMEM((2, ECH), jnp.int32),
            pltpu.VMEM((2, ECH), jnp.int32),
            pltpu.VMEM((ECH, DH), _F32),
            pltpu.VMEM((ECH, DH), _F32),
            pltpu.VMEM((ECH, DH), _F32),
            pltpu.VMEM((ECH, DH), _F32),
            pltpu.VMEM((ECH * DOUT,), _F32),
            pltpu.VMEM((ECH * DOUT,), _F32),
            pltpu.VMEM((EPW_T,), jnp.int32),
            pltpu.VMEM((EPW_T,), jnp.int32),
            pltpu.VMEM((EPW_T, DH), _F32),
            pltpu.VMEM((EPW_T, DH), _F32),
            pltpu.VMEM((EPW_T * DOUT,), _F32),
            pltpu.VMEM((DOUT, DH), _F32),
            pltpu.VMEM((16,), _F32),
            pltpu.SemaphoreType.DMA,
            pltpu.SemaphoreType.DMA,
            pltpu.SemaphoreType.DMA,
            pltpu.SemaphoreType.DMA,
            pltpu.SemaphoreType.DMA,
            pltpu.SemaphoreType.DMA,
        ],
    )
    def edge_k(p_ref, q_ref, src_ref, dst_ref, m2_ref, c2_ref, out_ref,
               idxb0, idxb1, pb0, pb1, qb0, qb1, ob0, ob1,
               idx_st, idx_dt, pbt, qbt, obt, m2v, c2v,
               semi0, semi1, semg0, semg1, semw0, semw1):
        c = lax.axis_index("c")
        s = lax.axis_index("s")
        w = c * NS + s
        pltpu.sync_copy(m2_ref, m2v)
        pltpu.sync_copy(c2_ref, c2v)
        m0 = [m2v[0, pl.ds(k * 16, 16)] for k in range(DH // 16)]
        m1 = [m2v[1, pl.ds(k * 16, 16)] for k in range(DH // 16)]
        c2vec = c2v[...]
        c2a = c2vec[0]
        c2b = c2vec[1]
        iot = lax.iota(jnp.int32, 16)
        base = w * EPW
        idxb = [idxb0, idxb1]
        pb = [pb0, pb1]
        qb = [qb0, qb1]
        ob = [ob0, ob1]
        semi = [semi0, semi1]
        semg = [semg0, semg1]
        semw = [semw0, semw1]

        def issue_idx(i, slot):
            off = pl.multiple_of(base + i * ECH, 16)
            pltpu.async_copy(src_ref.at[pl.ds(off, ECH)],
                             idxb[slot].at[0], semi[slot])
            pltpu.async_copy(dst_ref.at[pl.ds(off, ECH)],
                             idxb[slot].at[1], semi[slot])

        def drain_idx(slot):
            pltpu.make_async_copy(dst_ref.at[pl.ds(0, ECH)],
                                  idxb[slot].at[0], semi[slot]).wait()
            pltpu.make_async_copy(dst_ref.at[pl.ds(0, ECH)],
                                  idxb[slot].at[1], semi[slot]).wait()

        def issue_gathers(slot):
            pltpu.async_copy(p_ref.at[idxb[slot].at[0]], pb[slot], semg[slot])
            pltpu.async_copy(q_ref.at[idxb[slot].at[1]], qb[slot], semg[slot])

        def drain_gathers(slot):
            pltpu.make_async_copy(p_ref.at[pl.ds(0, ECH)], pb[slot],
                                  semg[slot]).wait()
            pltpu.make_async_copy(q_ref.at[pl.ds(0, ECH)], qb[slot],
                                  semg[slot]).wait()

        def compute(n, pbuf, qbuf, obuf):
            def ebody(e, carry):
                acc0 = jnp.zeros((16,), _F32)
                acc1 = jnp.zeros((16,), _F32)
                for k in range(DH // 16):
                    pv = pbuf[e, pl.ds(k * 16, 16)]
                    qv = qbuf[e, pl.ds(k * 16, 16)]
                    r = jnp.maximum(pv + qv, 0.0)
                    acc0 = acc0 + r * m0[k]
                    acc1 = acc1 + r * m1[k]
                s0 = jnp.sum(acc0) + c2a
                s1 = jnp.sum(acc1) + c2b
                vvals = jnp.where(iot == 0, s0, s1)
                vidx = e * DOUT + jnp.minimum(iot, 1)
                plsc.store_scatter(obuf, [vidx], vvals, mask=iot < DOUT)
                return carry

            lax.fori_loop(0, n, ebody, 0)

        issue_idx(0, 0)
        issue_idx(1, 1)
        drain_idx(0)
        issue_gathers(0)

        def body(j, carry):
            for b in range(2):
                i = j * 2 + b
                b2 = 1 - b

                @pl.when(i >= 2)
                def _():
                    pltpu.make_async_copy(ob[b], out_ref.at[pl.ds(0, ECH * DOUT)],
                                          semw[b]).wait()

                @pl.when(i + 1 < E_FC)
                def _():
                    drain_idx(b2)
                    issue_gathers(b2)

                drain_gathers(b)

                @pl.when(i + 2 < E_FC)
                def _():
                    issue_idx(i + 2, b)

                compute(ECH, pb[b], qb[b], ob[b])
                off = pl.multiple_of(base + i * ECH, 16)
                pltpu.async_copy(ob[b], out_ref.at[pl.ds(off * DOUT, ECH * DOUT)],
                                 semw[b])
            return carry

        lax.fori_loop(0, E_FC // 2, body, 0)
        pltpu.make_async_copy(ob[0], out_ref.at[pl.ds(0, ECH * DOUT)],
                              semw[0]).wait()
        pltpu.make_async_copy(ob[1], out_ref.at[pl.ds(0, ECH * DOUT)],
                              semw[1]).wait()
        off_t = base + E_FC * ECH
        pltpu.sync_copy(src_ref.at[pl.ds(off_t, EPW_T)], idx_st)
        pltpu.sync_copy(p_ref.at[idx_st], pbt)
        pltpu.sync_copy(dst_ref.at[pl.ds(off_t, EPW_T)], idx_dt)
        pltpu.sync_copy(q_ref.at[idx_dt], qbt)
        compute(EPW_T, pbt, qbt, obt)
        pltpu.sync_copy(obt, out_ref.at[pl.ds(off_t * DOUT, EPW_T * DOUT)])

    return edge_k(p_hbm, q_hbm, src, dst, m2t, c2pad)


_RB = 2000
_GRID = N // _RB


def _tc_a(x, dinv_bc, w1):
    def body(x_ref, d_ref, w_ref, g_ref):
        h = jnp.dot(x_ref[...], w_ref[...], preferred_element_type=_F32)
        g = d_ref[...] * h
        g_ref[0] = g[:, :HALF]
        g_ref[1] = g[:, HALF:]

    return pl.pallas_call(
        body,
        grid=(_GRID,),
        in_specs=[
            pl.BlockSpec((_RB, DIN), lambda i: (i, 0)),
            pl.BlockSpec((_RB, DH), lambda i: (i, 0)),
            pl.BlockSpec((DIN, DH), lambda i: (0, 0)),
        ],
        out_specs=pl.BlockSpec((NC, _RB, HALF), lambda i: (0, i, 0)),
        out_shape=jax.ShapeDtypeStruct((NC, N, HALF), _F32),
    )(x, dinv_bc, w1)


def _tc_b(s1, dinv_bc, b1, w2):
    def body(s_ref, d_ref, b_ref, w_ref, g_ref):
        scat = jnp.concatenate([s_ref[0], s_ref[1]], axis=1)
        act = jnp.maximum(d_ref[...] * scat + b_ref[0, :], 0.0)
        h2 = jnp.dot(act, w_ref[...], preferred_element_type=_F32)
        g = d_ref[...] * h2
        g_ref[0] = g[:, :HALF]
        g_ref[1] = g[:, HALF:]

    return pl.pallas_call(
        body,
        grid=(_GRID,),
        in_specs=[
            pl.BlockSpec((NC, _RB, HALF), lambda i: (0, i, 0)),
            pl.BlockSpec((_RB, DH), lambda i: (i, 0)),
            pl.BlockSpec((1, DH), lambda i: (0, 0)),
            pl.BlockSpec((DH, DH), lambda i: (0, 0)),
        ],
        out_specs=pl.BlockSpec((NC, _RB, HALF), lambda i: (0, i, 0)),
        out_shape=jax.ShapeDtypeStruct((NC, N, HALF), _F32),
    )(s1, dinv_bc, b1, w2)


def _tc_c(s2, dinv_bc, b2, m1, c1):
    def body(s_ref, d_ref, b_ref, m_ref, c_ref, p_ref, q_ref):
        scat = jnp.concatenate([s_ref[0], s_ref[1]], axis=1)
        h = d_ref[...] * scat + b_ref[0, :]
        p_ref[...] = jnp.dot(h, m_ref[:DH, :],
                             preferred_element_type=_F32) + c_ref[0, :]
        q_ref[...] = jnp.dot(h, m_ref[DH:, :], preferred_element_type=_F32)

    return pl.pallas_call(
        body,
        grid=(_GRID,),
        in_specs=[
            pl.BlockSpec((NC, _RB, HALF), lambda i: (0, i, 0)),
            pl.BlockSpec((_RB, DH), lambda i: (i, 0)),
            pl.BlockSpec((1, DH), lambda i: (0, 0)),
            pl.BlockSpec((2 * DH, DH), lambda i: (0, 0)),
            pl.BlockSpec((1, DH), lambda i: (0, 0)),
        ],
        out_specs=[
            pl.BlockSpec((_RB, DH), lambda i: (i, 0)),
            pl.BlockSpec((_RB, DH), lambda i: (i, 0)),
        ],
        out_shape=[
            jax.ShapeDtypeStruct((N, DH), _F32),
            jax.ShapeDtypeStruct((N, DH), _F32),
        ],
    )(s2, dinv_bc, b2, m1, c1)


def kernel(x, edge_index, W1, b1, W2, b2, M1, c1, M2, c2):
    ei = edge_index.astype(jnp.int32)
    src = ei[0]
    dst = ei[1]
    src2 = jnp.concatenate([src, src + N])
    b1r = b1.reshape(1, DH)
    b2r = b2.reshape(1, DH)
    c1r = c1.reshape(1, DH)
    m2t = M2.T.astype(_F32)
    c2pad = jnp.concatenate([c2, jnp.zeros((14,), _F32)])

    degp = _deg_call(dst).reshape(NW, N)
    dinv_bc = _tc_dinv(degp)
    g1 = _tc_a(x, dinv_bc, W1)
    s1 = _scatter_call(g1.reshape(NC * N, HALF), src2, dst)
    g2 = _tc_b(s1.reshape(NC, N, HALF), dinv_bc, b1r, W2)
    s2 = _scatter_call(g2.reshape(NC * N, HALF), src2, dst)
    p, q = _tc_c(s2.reshape(NC, N, HALF), dinv_bc, b2r, M1, c1r)
    out = _edge_call(p, q, src, dst, m2t, c2pad)
    return out.reshape(E, DOUT)


# speedup vs baseline: 11.5164x; 1.0919x over previous
import functools

import jax
import jax.numpy as jnp
from jax import lax
from jax.experimental import pallas as pl
from jax.experimental.pallas import tpu as pltpu
from jax.experimental.pallas import tpu_sc as plsc

N = 10000
E = 320000
DIN = 128
DH = 256
DOUT = 2
NC = 2
NS = 16
NW = NC * NS
HALF = DH // 2

EPW = E // NW
ECH = 96
E_FC = 104
EPW_T = EPW - E_FC * ECH
EPS = E // NS
EPS_FC = EPS // 128
EPS_T = EPS - EPS_FC * 128
RPW = 624
RT = N - NS * RPW

_F32 = jnp.float32


def _sc_mesh():
    return plsc.VectorSubcoreMesh(core_axis_name="c", subcore_axis_name="s")


_DCH = 2000


def _deg_call(dst_hbm):
    @functools.partial(
        pl.kernel,
        out_type=jax.ShapeDtypeStruct((NW * N,), _F32),
        mesh=_sc_mesh(),
        compiler_params=pltpu.CompilerParams(needs_layout_passes=False),
        scratch_types=[
            pltpu.VMEM((N,), _F32),
            pltpu.VMEM((_DCH,), jnp.int32),
        ],
    )
    def deg_k(dst_ref, out_ref, hist, dbuf):
        c = lax.axis_index("c")
        s = lax.axis_index("s")
        w = c * NS + s
        z = jnp.zeros((16,), _F32)
        one = jnp.ones((16,), _F32)

        def zbody(i, carry):
            hist[pl.ds(i * 16, 16)] = z
            return carry

        lax.fori_loop(0, N // 16, zbody, 0)
        base = w * EPW

        def chunk(ci, carry):
            off = pl.multiple_of(base + ci * _DCH, 16)
            pltpu.sync_copy(dst_ref.at[pl.ds(off, _DCH)], dbuf)

            def gbody(g, carry2):
                dv = dbuf[pl.ds(g * 16, 16)]
                plsc.addupdate_scatter(hist, [dv], one)
                return carry2

            lax.fori_loop(0, _DCH // 16, gbody, 0)
            return carry

        lax.fori_loop(0, EPW // _DCH, chunk, 0)
        pltpu.sync_copy(hist, out_ref.at[pl.ds(w * N, N)])

    return deg_k(dst_hbm)


def _scatter_call(gflat, src2, dst):
    @functools.partial(
        pl.kernel,
        out_type=jax.ShapeDtypeStruct((NC * N, HALF), _F32),
        mesh=_sc_mesh(),
        compiler_params=pltpu.CompilerParams(needs_layout_passes=False),
        scratch_types=[
            pltpu.VMEM_SHARED((N, HALF), _F32),
            pltpu.VMEM((2, 128), jnp.int32),
            pltpu.VMEM((2, 128), jnp.int32),
            pltpu.VMEM((2, 128), jnp.int32),
            pltpu.VMEM((2, 128), jnp.int32),
            pltpu.VMEM((128, HALF), _F32),
            pltpu.VMEM((128, HALF), _F32),
            pltpu.VMEM((EPS_T,), jnp.int32),
            pltpu.VMEM((EPS_T,), jnp.int32),
            pltpu.VMEM((EPS_T, HALF), _F32),
            pltpu.SemaphoreType.DMA,
            pltpu.SemaphoreType.DMA,
            pltpu.SemaphoreType.DMA,
            pltpu.SemaphoreType.DMA,
            pltpu.SemaphoreType.DMA,
            pltpu.SemaphoreType.DMA,
            pltpu.SemaphoreType.DMA,
        ],
    )
    def scat_k(g_ref, src_ref, dst_ref, out_ref,
               acc, idx0, idx1, idx2, idx3, rows0, rows1,
               idx_st, idx_dt, rows_t,
               semi0, semi1, semi2, semi3, semg, sems0, sems1):
        c = lax.axis_index("c")
        s = lax.axis_index("s")
        r0 = s * RPW
        idx_bufs = [idx0, idx1, idx2, idx3]
        semi = [semi0, semi1, semi2, semi3]
        rows = [rows0, rows1]
        sems = [sems0, sems1]
        ebase = s * EPS

        def issue_idx(i, slot):
            off = pl.multiple_of(ebase + i * 128, 32)
            pltpu.async_copy(src_ref.at[pl.ds(c * E + off, 128)],
                             idx_bufs[slot].at[0], semi[slot])
            pltpu.async_copy(dst_ref.at[pl.ds(off, 128)],
                             idx_bufs[slot].at[1], semi[slot])

        def drain_idx(slot):
            pltpu.make_async_copy(dst_ref.at[pl.ds(0, 128)],
                                  idx_bufs[slot].at[0], semi[slot]).wait()
            pltpu.make_async_copy(dst_ref.at[pl.ds(0, 128)],
                                  idx_bufs[slot].at[1], semi[slot]).wait()

        pltpu.sync_copy(g_ref.at[pl.ds(c * N + r0, RPW)], acc.at[pl.ds(r0, RPW)])

        @pl.when(s == 0)
        def _():
            pltpu.sync_copy(g_ref.at[pl.ds(c * N + NS * RPW, RT)],
                            acc.at[pl.ds(NS * RPW, RT)])

        plsc.subcore_barrier()
        issue_idx(0, 0)
        issue_idx(1, 1)

        def body(j, carry):
            for b in range(4):
                i = j * 4 + b

                @pl.when(i >= 2)
                def _():
                    pltpu.make_async_copy(g_ref.at[pl.ds(0, 128)],
                                          rows[b % 2], sems[b % 2]).wait()

                @pl.when(i + 2 < EPS_FC)
                def _():
                    issue_idx(i + 2, (b + 2) % 4)

                drain_idx(b)
                pltpu.async_copy(g_ref.at[idx_bufs[b].at[0]], rows[b % 2],
                                 semg).wait()
                pltpu.async_copy(rows[b % 2], acc.at[idx_bufs[b].at[1]],
                                 sems[b % 2], add=True)
            return carry

        lax.fori_loop(0, EPS_FC // 4, body, 0)
        pltpu.make_async_copy(g_ref.at[pl.ds(0, 128)], rows[0], sems[0]).wait()
        pltpu.make_async_copy(g_ref.at[pl.ds(0, 128)], rows[1], sems[1]).wait()
        off_t = ebase + EPS_FC * 128
        pltpu.sync_copy(src_ref.at[pl.ds(c * E + off_t, EPS_T)], idx_st)
        pltpu.sync_copy(g_ref.at[idx_st], rows_t)
        pltpu.sync_copy(dst_ref.at[pl.ds(off_t, EPS_T)], idx_dt)
        pltpu.sync_copy(rows_t, acc.at[idx_dt], add=True)
        plsc.subcore_barrier()
        pltpu.sync_copy(acc.at[pl.ds(r0, RPW)],
                        out_ref.at[pl.ds(c * N + r0, RPW)])

        @pl.when(s == 0)
        def _():
            pltpu.sync_copy(acc.at[pl.ds(NS * RPW, RT)],
                            out_ref.at[pl.ds(c * N + NS * RPW, RT)])

    return scat_k(gflat, src2, dst)


def _edge_call(p_hbm, q_hbm, src, dst, m2t, c2pad):
    @functools.partial(
        pl.kernel,
        out_type=jax.ShapeDtypeStruct((E * DOUT,), _F32),
        mesh=_sc_mesh(),
        compiler_params=pltpu.CompilerParams(needs_layout_passes=False),
        scratch_types=[
            pltpu.VMEM((2, ECH), jnp.int32),
            pltpu.VMEM((2, ECH), jnp.int32),
            pltpu.VMEM((ECH, DH), _F32),
            pltpu.VMEM((ECH, DH), _F32),
            pltpu.VMEM((ECH, DH), _F32),
            pltpu.VMEM((ECH, DH), _F32),
            pltpu.VMEM((ECH * DOUT,), _F32),
            pltpu.VMEM((ECH * DOUT,), _F32),
            pltpu.VMEM((EPW_T,), jnp.int32),
            pltpu.VMEM((EPW_T,), jnp.int32),
            pltpu.VMEM((EPW_T, DH), _F32),
            pltpu.VMEM((EPW_T, DH), _F32),
            pltpu.VMEM((EPW_T * DOUT,), _F32),
            pltpu.VMEM((DOUT, DH), _F32),
            pltpu.VMEM((16,), _F32),
            pltpu.SemaphoreType.DMA,
            pltpu.SemaphoreType.DMA,
            pltpu.SemaphoreType.DMA,
            pltpu.SemaphoreType.DMA,
            pltpu.SemaphoreType.DMA,
            pltpu.SemaphoreType.DMA,
        ],
    )
    def edge_k(p_ref, q_ref, src_ref, dst_ref, m2_ref, c2_ref, out_ref,
               idxb0, idxb1, pb0, pb1, qb0, qb1, ob0, ob1,
               idx_st, idx_dt, pbt, qbt, obt, m2v, c2v,
               semi0, semi1, semg0, semg1, semw0, semw1):
        c = lax.axis_index("c")
        s = lax.axis_index("s")
        w = c * NS + s
        pltpu.sync_copy(m2_ref, m2v)
        pltpu.sync_copy(c2_ref, c2v)
        m0 = [m2v[0, pl.ds(k * 16, 16)] for k in range(DH // 16)]
        m1 = [m2v[1, pl.ds(k * 16, 16)] for k in range(DH // 16)]
        c2vec = c2v[...]
        c2a = c2vec[0]
        c2b = c2vec[1]
        iot = lax.iota(jnp.int32, 16)
        base = w * EPW
        idxb = [idxb0, idxb1]
        pb = [pb0, pb1]
        qb = [qb0, qb1]
        ob = [ob0, ob1]
        semi = [semi0, semi1]
        semg = [semg0, semg1]
        semw = [semw0, semw1]

        def issue_idx(i, slot):
            off = pl.multiple_of(base + i * ECH, 16)
            pltpu.async_copy(src_ref.at[pl.ds(off, ECH)],
                             idxb[slot].at[0], semi[slot])
            pltpu.async_copy(dst_ref.at[pl.ds(off, ECH)],
                             idxb[slot].at[1], semi[slot])

        def drain_idx(slot):
            pltpu.make_async_copy(dst_ref.at[pl.ds(0, ECH)],
                                  idxb[slot].at[0], semi[slot]).wait()
            pltpu.make_async_copy(dst_ref.at[pl.ds(0, ECH)],
                                  idxb[slot].at[1], semi[slot]).wait()

        def issue_gathers(slot):
            pltpu.async_copy(p_ref.at[idxb[slot].at[0]], pb[slot], semg[slot])
            pltpu.async_copy(q_ref.at[idxb[slot].at[1]], qb[slot], semg[slot])

        def drain_gathers(slot):
            pltpu.make_async_copy(p_ref.at[pl.ds(0, ECH)], pb[slot],
                                  semg[slot]).wait()
            pltpu.make_async_copy(q_ref.at[pl.ds(0, ECH)], qb[slot],
                                  semg[slot]).wait()

        c2init0 = jnp.where(iot == 0, c2a, 0.0)
        c2init1 = jnp.where(iot == 0, c2b, 0.0)

        def compute(npairs, pbuf, qbuf, obuf):
            def ebody(eh, carry):
                for u in range(2):
                    e = eh * 2 + u
                    acc0 = c2init0
                    acc1 = c2init1
                    for k in range(DH // 16):
                        pv = pbuf[e, pl.ds(k * 16, 16)]
                        qv = qbuf[e, pl.ds(k * 16, 16)]
                        r = jnp.maximum(pv + qv, 0.0)
                        acc0 = acc0 + r * m0[k]
                        acc1 = acc1 + r * m1[k]
                    vvals = jnp.where(iot == 0, jnp.sum(acc0), jnp.sum(acc1))
                    vidx = e * DOUT + jnp.minimum(iot, 1)
                    plsc.store_scatter(obuf, [vidx], vvals, mask=iot < DOUT)
                return carry

            lax.fori_loop(0, npairs, ebody, 0)

        issue_idx(0, 0)
        issue_idx(1, 1)
        drain_idx(0)
        issue_gathers(0)

        def body(j, carry):
            for b in range(2):
                i = j * 2 + b
                b2 = 1 - b

                @pl.when(i >= 2)
                def _():
                    pltpu.make_async_copy(ob[b], out_ref.at[pl.ds(0, ECH * DOUT)],
                                          semw[b]).wait()

                @pl.when(i + 1 < E_FC)
                def _():
                    drain_idx(b2)
                    issue_gathers(b2)

                drain_gathers(b)

                @pl.when(i + 2 < E_FC)
                def _():
                    issue_idx(i + 2, b)

                compute(ECH // 2, pb[b], qb[b], ob[b])
                off = pl.multiple_of(base + i * ECH, 16)
                pltpu.async_copy(ob[b], out_ref.at[pl.ds(off * DOUT, ECH * DOUT)],
                                 semw[b])
            return carry

        lax.fori_loop(0, E_FC // 2, body, 0)
        pltpu.make_async_copy(ob[0], out_ref.at[pl.ds(0, ECH * DOUT)],
                              semw[0]).wait()
        pltpu.make_async_copy(ob[1], out_ref.at[pl.ds(0, ECH * DOUT)],
                              semw[1]).wait()
        off_t = base + E_FC * ECH
        pltpu.sync_copy(src_ref.at[pl.ds(off_t, EPW_T)], idx_st)
        pltpu.sync_copy(p_ref.at[idx_st], pbt)
        pltpu.sync_copy(dst_ref.at[pl.ds(off_t, EPW_T)], idx_dt)
        pltpu.sync_copy(q_ref.at[idx_dt], qbt)
        compute(EPW_T // 2, pbt, qbt, obt)
        pltpu.sync_copy(obt, out_ref.at[pl.ds(off_t * DOUT, EPW_T * DOUT)])

    return edge_k(p_hbm, q_hbm, src, dst, m2t, c2pad)


_RB = 2000
_GRID = N // _RB


def _tc_a(x, degp, w1):
    def body(x_ref, d_ref, w_ref, g_ref, dinv_ref):
        deg = jnp.sum(d_ref[...], axis=0, keepdims=True) + 1.0
        dinv = lax.rsqrt(deg)
        dinv_bc = jnp.broadcast_to(jnp.transpose(dinv), (N, DH))
        h = jnp.dot(x_ref[...], w_ref[...], preferred_element_type=_F32)
        g = dinv_bc * h
        g_ref[0] = g[:, :HALF]
        g_ref[1] = g[:, HALF:]
        dinv_ref[...] = dinv_bc

    return pl.pallas_call(
        body,
        out_shape=[
            jax.ShapeDtypeStruct((NC, N, HALF), _F32),
            jax.ShapeDtypeStruct((N, DH), _F32),
        ],
    )(x, degp, w1)


def _tc_b(s1, dinv_bc, b1, w2):
    def body(s_ref, d_ref, b_ref, w_ref, g_ref):
        scat = jnp.concatenate([s_ref[0], s_ref[1]], axis=1)
        act = jnp.maximum(d_ref[...] * scat + b_ref[0, :], 0.0)
        h2 = jnp.dot(act, w_ref[...], preferred_element_type=_F32)
        g = d_ref[...] * h2
        g_ref[0] = g[:, :HALF]
        g_ref[1] = g[:, HALF:]

    return pl.pallas_call(
        body,
        grid=(_GRID,),
        in_specs=[
            pl.BlockSpec((NC, _RB, HALF), lambda i: (0, i, 0)),
            pl.BlockSpec((_RB, DH), lambda i: (i, 0)),
            pl.BlockSpec((1, DH), lambda i: (0, 0)),
            pl.BlockSpec((DH, DH), lambda i: (0, 0)),
        ],
        out_specs=pl.BlockSpec((NC, _RB, HALF), lambda i: (0, i, 0)),
        out_shape=jax.ShapeDtypeStruct((NC, N, HALF), _F32),
    )(s1, dinv_bc, b1, w2)


def _tc_c(s2, dinv_bc, b2, m1, c1):
    def body(s_ref, d_ref, b_ref, m_ref, c_ref, p_ref, q_ref):
        scat = jnp.concatenate([s_ref[0], s_ref[1]], axis=1)
        h = d_ref[...] * scat + b_ref[0, :]
        p_ref[...] = jnp.dot(h, m_ref[:DH, :],
                             preferred_element_type=_F32) + c_ref[0, :]
        q_ref[...] = jnp.dot(h, m_ref[DH:, :], preferred_element_type=_F32)

    return pl.pallas_call(
        body,
        grid=(_GRID,),
        in_specs=[
            pl.BlockSpec((NC, _RB, HALF), lambda i: (0, i, 0)),
            pl.BlockSpec((_RB, DH), lambda i: (i, 0)),
            pl.BlockSpec((1, DH), lambda i: (0, 0)),
            pl.BlockSpec((2 * DH, DH), lambda i: (0, 0)),
            pl.BlockSpec((1, DH), lambda i: (0, 0)),
        ],
        out_specs=[
            pl.BlockSpec((_RB, DH), lambda i: (i, 0)),
            pl.BlockSpec((_RB, DH), lambda i: (i, 0)),
        ],
        out_shape=[
            jax.ShapeDtypeStruct((N, DH), _F32),
            jax.ShapeDtypeStruct((N, DH), _F32),
        ],
    )(s2, dinv_bc, b2, m1, c1)


def kernel(x, edge_index, W1, b1, W2, b2, M1, c1, M2, c2):
    ei = edge_index.astype(jnp.int32)
    src = ei[0]
    dst = ei[1]
    src2 = jnp.concatenate([src, src + N])
    b1r = b1.reshape(1, DH)
    b2r = b2.reshape(1, DH)
    c1r = c1.reshape(1, DH)
    m2t = M2.T.astype(_F32)
    c2pad = jnp.concatenate([c2, jnp.zeros((14,), _F32)])

    degp = _deg_call(dst).reshape(NW, N)
    g1, dinv_bc = _tc_a(x, degp, W1)
    s1 = _scatter_call(g1.reshape(NC * N, HALF), src2, dst)
    g2 = _tc_b(s1.reshape(NC, N, HALF), dinv_bc, b1r, W2)
    s2 = _scatter_call(g2.reshape(NC * N, HALF), src2, dst)
    p, q = _tc_c(s2.reshape(NC, N, HALF), dinv_bc, b2r, M1, c1r)
    out = _edge_call(p, q, src, dst, m2t, c2pad)
    return out.reshape(E, DOUT)
